# baseline (speedup 1.0000x reference)
import functools

import jax
import jax.numpy as jnp
from jax import lax
from jax.experimental import pallas as pl
from jax.experimental.pallas import tpu as pltpu
from jax.experimental.pallas import tpu_sc as plsc

N = 10000
E = 320000
D_IN = 128
D_H = 64

NC = 2
NS = 16
NW = NC * NS
CH = 128
K = 80
EPW = K * CH
EPAD = NW * EPW
N2 = 10240
RPT = N // NS
ZPT = N2 // NS

_mesh = plsc.VectorSubcoreMesh(core_axis_name="c", subcore_axis_name="s")
_sc_params = pltpu.CompilerParams(use_tc_tiling_on_sc=False)


@functools.partial(
    pl.kernel,
    out_type=jax.ShapeDtypeStruct((NC, N2), jnp.float32),
    mesh=_mesh,
    compiler_params=_sc_params,
    scratch_types=[
        pltpu.VMEM((K, CH), jnp.int32),
        pltpu.VMEM((CH,), jnp.float32),
        pltpu.VMEM((ZPT,), jnp.float32),
        pltpu.VMEM_SHARED((N2,), jnp.float32),
        pltpu.SemaphoreType.DMA,
    ],
)
def _deg_kernel(dst_hbm, ones_hbm, zz_hbm, out_hbm, dstv, onesv, zbuf, hist, sem):
    c = lax.axis_index("c")
    s = lax.axis_index("s")
    wid = s * NC + c
    pltpu.sync_copy(dst_hbm.at[wid], dstv)
    pltpu.sync_copy(ones_hbm, onesv)
    pltpu.sync_copy(zz_hbm, zbuf)
    pltpu.sync_copy(zbuf, hist.at[pl.ds(s * ZPT, ZPT)])
    plsc.subcore_barrier()

    def body(k, carry):
        pltpu.async_copy(onesv, hist.at[dstv.at[k]], sem, add=True).wait()
        return carry

    lax.fori_loop(0, K, body, 0)
    plsc.subcore_barrier()
    pltpu.sync_copy(hist.at[pl.ds(s * ZPT, ZPT)], out_hbm.at[c, pl.ds(s * ZPT, ZPT)])


@functools.partial(
    pl.kernel,
    out_type=jax.ShapeDtypeStruct((NC, N2, D_H), jnp.float32),
    mesh=_mesh,
    compiler_params=_sc_params,
    scratch_types=[
        pltpu.VMEM((K, CH), jnp.int32),
        pltpu.VMEM((K, CH), jnp.int32),
        pltpu.VMEM((8, CH, D_H), jnp.float32),
        pltpu.VMEM_SHARED((N2, D_H), jnp.float32),
        [pltpu.SemaphoreType.DMA] * 8,
        [pltpu.SemaphoreType.DMA] * 8,
    ],
)
def _edge_kernel(g_hbm, src_hbm, dst_hbm, zz_hbm, out_hbm,
                 srcv, dstv, rows, acc, gsems, ssems):
    c = lax.axis_index("c")
    s = lax.axis_index("s")
    wid = s * NC + c
    pltpu.sync_copy(src_hbm.at[wid], srcv)
    pltpu.sync_copy(dst_hbm.at[wid], dstv)
    pltpu.sync_copy(zz_hbm, rows.at[0])
    for j in range(ZPT // CH):
        pltpu.sync_copy(rows.at[0], acc.at[pl.ds(s * ZPT + j * CH, CH)])
    plsc.subcore_barrier()

    def body(i, carry):
        gds = [pltpu.async_copy(g_hbm.at[srcv.at[8 * i + b]], rows.at[b],
                                gsems[b]) for b in range(8)]
        sds = []
        for b in range(8):
            gds[b].wait()
            sds.append(pltpu.async_copy(rows.at[b], acc.at[dstv.at[8 * i + b]],
                                        ssems[b], add=True))
        for sdd in sds:
            sdd.wait()
        return carry

    lax.fori_loop(0, K // 8, body, 0)
    plsc.subcore_barrier()
    pltpu.sync_copy(acc.at[pl.ds(s * ZPT, ZPT)],
                    out_hbm.at[c, pl.ds(s * ZPT, ZPT)])


def _prep_body(x_ref, w1_ref, degp_ref, g_ref, dinv_ref):
    deg = degp_ref[0] + degp_ref[1] + 1.0
    dinv = lax.rsqrt(deg)
    h = jnp.dot(x_ref[...], w1_ref[...], preferred_element_type=jnp.float32)
    g_ref[...] = h * dinv
    dinv_ref[...] = dinv


def _mid_body(part_ref, g1_ref, dinv_ref, b1_ref, gm1_ref, bt1_ref, w2_ref,
              g2_ref):
    ssum = part_ref[0, :N, :] + part_ref[1, :N, :] + g1_ref[...]
    out1 = ssum * dinv_ref[...] + b1_ref[...]
    mu = jnp.mean(out1, axis=0, keepdims=True)
    var = jnp.mean((out1 - mu) ** 2, axis=0, keepdims=True)
    hbn = (out1 - mu) * lax.rsqrt(var + 1e-5) * gm1_ref[...] + bt1_ref[...]
    hrelu = jnp.maximum(hbn, 0.0)
    h2 = jnp.dot(hrelu, w2_ref[...], preferred_element_type=jnp.float32)
    g2_ref[...] = h2 * dinv_ref[...]


def _fin_body(part_ref, g2_ref, dinv_ref, b2_ref, gm2_ref, bt2_ref, out_ref):
    ssum = part_ref[0, :N, :] + part_ref[1, :N, :] + g2_ref[...]
    out2 = ssum * dinv_ref[...] + b2_ref[...]
    mu = jnp.mean(out2, axis=0, keepdims=True)
    var = jnp.mean((out2 - mu) ** 2, axis=0, keepdims=True)
    out_ref[...] = (out2 - mu) * lax.rsqrt(var + 1e-5) * gm2_ref[...] + bt2_ref[...]


_prep_call = pl.pallas_call(
    _prep_body,
    out_shape=[jax.ShapeDtypeStruct((N, D_H), jnp.float32),
               jax.ShapeDtypeStruct((N, 1), jnp.float32)],
)

_mid_call = pl.pallas_call(
    _mid_body,
    out_shape=jax.ShapeDtypeStruct((N, D_H), jnp.float32),
)

_fin_call = pl.pallas_call(
    _fin_body,
    out_shape=jax.ShapeDtypeStruct((N, D_H), jnp.float32),
)


def kernel(x, edge_index, W1, b1, gamma1, beta1, W2, b2, gamma2, beta2):
    pad = EPAD - E
    src = jnp.concatenate([edge_index[0], jnp.zeros((pad,), jnp.int32)])
    dst = jnp.concatenate([edge_index[1], jnp.full((pad,), N, jnp.int32)])
    srcp = src.reshape(NW, K, CH)
    dstp = dst.reshape(NW, K, CH)

    ones_c = jnp.ones((CH,), jnp.float32)
    zz1 = jnp.zeros((ZPT,), jnp.float32)
    zz2 = jnp.zeros((CH, D_H), jnp.float32)

    degp = _deg_kernel(dstp, ones_c, zz1)
    degp2 = degp[:, :N, None]

    g1, dinv = _prep_call(x, W1, degp2)

    part1 = _edge_kernel(g1, srcp, dstp, zz2)
    g2 = _mid_call(part1, g1, dinv, b1[None, :], gamma1[None, :],
                   beta1[None, :], W2)

    part2 = _edge_kernel(g2, srcp, dstp, zz2)
    out = _fin_call(part2, g2, dinv, b2[None, :], gamma2[None, :],
                    beta2[None, :])
    return out


# speedup vs baseline: 1.5413x; 1.5413x over previous
import functools

import jax
import jax.numpy as jnp
from jax import lax
from jax.experimental import pallas as pl
from jax.experimental.pallas import tpu as pltpu
from jax.experimental.pallas import tpu_sc as plsc

N = 10000
E = 320000
D_IN = 128
D_H = 64

NC = 2
NS = 16
NW = NC * NS
CH = 128
K = 80
EPW = K * CH
EPAD = NW * EPW
N2 = 10240
RPT = N // NS
ZPT = N2 // NS

_mesh = plsc.VectorSubcoreMesh(core_axis_name="c", subcore_axis_name="s")
_sc_params = pltpu.CompilerParams(use_tc_tiling_on_sc=False)


@functools.partial(
    pl.kernel,
    out_type=jax.ShapeDtypeStruct((NC, N2), jnp.float32),
    mesh=_mesh,
    compiler_params=_sc_params,
    scratch_types=[
        pltpu.VMEM((K, CH), jnp.int32),
        pltpu.VMEM((CH,), jnp.float32),
        pltpu.VMEM((ZPT,), jnp.float32),
        pltpu.VMEM_SHARED((N2,), jnp.float32),
        pltpu.SemaphoreType.DMA,
    ],
)
def _deg_kernel(dst_hbm, ones_hbm, zz_hbm, out_hbm, dstv, onesv, zbuf, hist, sem):
    c = lax.axis_index("c")
    s = lax.axis_index("s")
    wid = s * NC + c
    pltpu.sync_copy(dst_hbm.at[wid], dstv)
    pltpu.sync_copy(ones_hbm, onesv)
    pltpu.sync_copy(zz_hbm, zbuf)
    pltpu.sync_copy(zbuf, hist.at[pl.ds(s * ZPT, ZPT)])
    plsc.subcore_barrier()

    def body(k, carry):
        pltpu.async_copy(onesv, hist.at[dstv.at[k]], sem, add=True).wait()
        return carry

    lax.fori_loop(0, K, body, 0)
    plsc.subcore_barrier()
    pltpu.sync_copy(hist.at[pl.ds(s * ZPT, ZPT)], out_hbm.at[c, pl.ds(s * ZPT, ZPT)])


NP = 5


@functools.partial(
    pl.kernel,
    out_type=jax.ShapeDtypeStruct((NC, NP, N2, D_H), jnp.bfloat16),
    mesh=_mesh,
    compiler_params=_sc_params,
    scratch_types=[
        pltpu.VMEM((K, CH), jnp.int32),
        pltpu.VMEM((K, CH), jnp.int32),
        pltpu.VMEM((8, CH, D_H), jnp.bfloat16),
        pltpu.VMEM((CH, D_H), jnp.bfloat16),
        pltpu.VMEM_SHARED((N2, D_H), jnp.bfloat16),
        pltpu.VMEM_SHARED((N, D_H), jnp.bfloat16),
        [pltpu.SemaphoreType.DMA] * 8,
        [pltpu.SemaphoreType.DMA] * 8,
    ],
)
def _edge_kernel(g_hbm, src_hbm, dst_hbm, zz_hbm, out_hbm,
                 srcv, dstv, rows, zbuf, acc, gtab, gsems, ssems):
    c = lax.axis_index("c")
    s = lax.axis_index("s")
    wid = s * NC + c
    pltpu.sync_copy(src_hbm.at[wid], srcv)
    pltpu.sync_copy(dst_hbm.at[wid], dstv)
    pltpu.sync_copy(g_hbm.at[pl.ds(s * RPT, RPT)], gtab.at[pl.ds(s * RPT, RPT)])
    pltpu.sync_copy(zz_hbm, zbuf)
    for j in range(ZPT // CH):
        pltpu.sync_copy(zbuf, acc.at[pl.ds(s * ZPT + j * CH, CH)])
    plsc.subcore_barrier()

    def body(i, carry):
        gds = [pltpu.async_copy(gtab.at[srcv.at[8 * i + b]], rows.at[b],
                                gsems[b]) for b in range(8)]
        sds = []
        for b in range(8):
            gds[b].wait()
            sds.append(pltpu.async_copy(rows.at[b], acc.at[dstv.at[8 * i + b]],
                                        ssems[b], add=True))
        for sdd in sds:
            sdd.wait()
        return carry

    KPP = K // NP
    for p in range(NP):
        lax.fori_loop(p * KPP // 8, (p + 1) * KPP // 8, body, 0)
        plsc.subcore_barrier()
        pltpu.sync_copy(acc.at[pl.ds(s * ZPT, ZPT)],
                        out_hbm.at[c, p, pl.ds(s * ZPT, ZPT)])
        if p + 1 < NP:
            for j in range(ZPT // CH):
                pltpu.sync_copy(zbuf, acc.at[pl.ds(s * ZPT + j * CH, CH)])
            plsc.subcore_barrier()


def _prep_body(x_ref, w1_ref, degp_ref, g_ref, gf_ref, dinv_ref):
    deg = degp_ref[0] + degp_ref[1] + 1.0
    dinv = lax.rsqrt(deg)
    h = jnp.dot(x_ref[...], w1_ref[...], preferred_element_type=jnp.float32)
    gf = h * dinv
    g_ref[...] = gf.astype(jnp.bfloat16)
    gf_ref[...] = gf
    dinv_ref[...] = dinv


def _psum(part_ref):
    acc = jnp.zeros((N, D_H), jnp.float32)
    for c in range(NC):
        for p in range(NP):
            acc = acc + part_ref[c, p, :N, :].astype(jnp.float32)
    return acc


def _mid_body(part_ref, g1_ref, dinv_ref, b1_ref, gm1_ref, bt1_ref, w2_ref,
              g2_ref, g2f_ref):
    psum = _psum(part_ref)
    out1 = (psum + g1_ref[...]) * dinv_ref[...] + b1_ref[...]
    mu = jnp.mean(out1, axis=0, keepdims=True)
    var = jnp.mean((out1 - mu) ** 2, axis=0, keepdims=True)
    hbn = (out1 - mu) * lax.rsqrt(var + 1e-5) * gm1_ref[...] + bt1_ref[...]
    hrelu = jnp.maximum(hbn, 0.0)
    h2 = jnp.dot(hrelu, w2_ref[...], preferred_element_type=jnp.float32)
    g2f = h2 * dinv_ref[...]
    g2_ref[...] = g2f.astype(jnp.bfloat16)
    g2f_ref[...] = g2f


def _fin_body(part_ref, g2_ref, dinv_ref, b2_ref, gm2_ref, bt2_ref, out_ref):
    psum = _psum(part_ref)
    out2 = (psum + g2_ref[...]) * dinv_ref[...] + b2_ref[...]
    mu = jnp.mean(out2, axis=0, keepdims=True)
    var = jnp.mean((out2 - mu) ** 2, axis=0, keepdims=True)
    out_ref[...] = (out2 - mu) * lax.rsqrt(var + 1e-5) * gm2_ref[...] + bt2_ref[...]


_prep_call = pl.pallas_call(
    _prep_body,
    out_shape=[jax.ShapeDtypeStruct((N, D_H), jnp.bfloat16),
               jax.ShapeDtypeStruct((N, D_H), jnp.float32),
               jax.ShapeDtypeStruct((N, 1), jnp.float32)],
)

_mid_call = pl.pallas_call(
    _mid_body,
    out_shape=[jax.ShapeDtypeStruct((N, D_H), jnp.bfloat16),
               jax.ShapeDtypeStruct((N, D_H), jnp.float32)],
)

_fin_call = pl.pallas_call(
    _fin_body,
    out_shape=jax.ShapeDtypeStruct((N, D_H), jnp.float32),
)


def kernel(x, edge_index, W1, b1, gamma1, beta1, W2, b2, gamma2, beta2):
    pad = EPAD - E
    src = jnp.concatenate([edge_index[0], jnp.zeros((pad,), jnp.int32)])
    dst = jnp.concatenate([edge_index[1], jnp.full((pad,), N, jnp.int32)])
    srcp = src.reshape(NW, K, CH)
    dstp = dst.reshape(NW, K, CH)

    ones_c = jnp.ones((CH,), jnp.float32)
    zz1 = jnp.zeros((ZPT,), jnp.float32)
    zz2 = jnp.zeros((CH, D_H), jnp.bfloat16)

    degp = _deg_kernel(dstp, ones_c, zz1)
    degp2 = degp[:, :N, None]

    g1, g1f, dinv = _prep_call(x, W1, degp2)

    part1 = _edge_kernel(g1, srcp, dstp, zz2)
    g2, g2f = _mid_call(part1, g1f, dinv, b1[None, :], gamma1[None, :],
                        beta1[None, :], W2)

    part2 = _edge_kernel(g2, srcp, dstp, zz2)
    out = _fin_call(part2, g2f, dinv, b2[None, :], gamma2[None, :],
                    beta2[None, :])
    return out


# speedup vs baseline: 1.7463x; 1.1330x over previous
import functools

import jax
import jax.numpy as jnp
from jax import lax
from jax.experimental import pallas as pl
from jax.experimental.pallas import tpu as pltpu
from jax.experimental.pallas import tpu_sc as plsc

N = 10000
E = 320000
D_IN = 128
D_H = 64

NC = 2
NS = 16
NW = NC * NS
CH = 128
K = 80
EPW = K * CH
EPAD = NW * EPW
N2 = 10240
RPT = N // NS
ZPT = N2 // NS

_mesh = plsc.VectorSubcoreMesh(core_axis_name="c", subcore_axis_name="s")
_sc_params = pltpu.CompilerParams(use_tc_tiling_on_sc=False)


@functools.partial(
    pl.kernel,
    out_type=jax.ShapeDtypeStruct((NC, N2), jnp.float32),
    mesh=_mesh,
    compiler_params=_sc_params,
    scratch_types=[
        pltpu.VMEM((K, CH), jnp.int32),
        pltpu.VMEM((CH,), jnp.float32),
        pltpu.VMEM((ZPT,), jnp.float32),
        pltpu.VMEM_SHARED((N2,), jnp.float32),
        pltpu.SemaphoreType.DMA,
    ],
)
def _deg_kernel(dst_hbm, ones_hbm, zz_hbm, out_hbm, dstv, onesv, zbuf, hist, sem):
    c = lax.axis_index("c")
    s = lax.axis_index("s")
    wid = s * NC + c
    pltpu.sync_copy(dst_hbm.at[wid], dstv)
    pltpu.sync_copy(ones_hbm, onesv)
    pltpu.sync_copy(zz_hbm, zbuf)
    pltpu.sync_copy(zbuf, hist.at[pl.ds(s * ZPT, ZPT)])
    plsc.subcore_barrier()

    def body(k, carry):
        pltpu.async_copy(onesv, hist.at[dstv.at[k]], sem, add=True).wait()
        return carry

    lax.fori_loop(0, K, body, 0)
    plsc.subcore_barrier()
    pltpu.sync_copy(hist.at[pl.ds(s * ZPT, ZPT)], out_hbm.at[c, pl.ds(s * ZPT, ZPT)])


NP = 5


@functools.partial(
    pl.kernel,
    out_type=jax.ShapeDtypeStruct((NC, NP, N2, D_H), jnp.bfloat16),
    mesh=_mesh,
    compiler_params=_sc_params,
    scratch_types=[
        pltpu.VMEM((K, CH), jnp.int32),
        pltpu.VMEM((K, CH), jnp.int32),
        pltpu.VMEM((8, CH, D_H), jnp.bfloat16),
        pltpu.VMEM((CH, D_H), jnp.bfloat16),
        pltpu.VMEM_SHARED((N2, D_H), jnp.bfloat16),
        pltpu.VMEM_SHARED((N, D_H), jnp.bfloat16),
        [pltpu.SemaphoreType.DMA] * 8,
        [pltpu.SemaphoreType.DMA] * 8,
    ],
)
def _edge_kernel(g_hbm, src_hbm, dst_hbm, zz_hbm, out_hbm,
                 srcv, dstv, rows, zbuf, acc, gtab, gsems, ssems):
    c = lax.axis_index("c")
    s = lax.axis_index("s")
    wid = s * NC + c
    pltpu.sync_copy(src_hbm.at[wid], srcv)
    pltpu.sync_copy(dst_hbm.at[wid], dstv)
    pltpu.sync_copy(g_hbm.at[pl.ds(s * RPT, RPT)], gtab.at[pl.ds(s * RPT, RPT)])
    pltpu.sync_copy(zz_hbm, zbuf)
    for j in range(ZPT // CH):
        pltpu.sync_copy(zbuf, acc.at[pl.ds(s * ZPT + j * CH, CH)])
    plsc.subcore_barrier()

    def body(i, carry):
        gds = [pltpu.async_copy(gtab.at[srcv.at[8 * i + b]], rows.at[b],
                                gsems[b]) for b in range(8)]
        sds = []
        for b in range(8):
            gds[b].wait()
            sds.append(pltpu.async_copy(rows.at[b], acc.at[dstv.at[8 * i + b]],
                                        ssems[b], add=True))
        for sdd in sds:
            sdd.wait()
        return carry

    KPP = K // NP
    for p in range(NP):
        lax.fori_loop(p * KPP // 8, (p + 1) * KPP // 8, body, 0)
        plsc.subcore_barrier()
        pltpu.sync_copy(acc.at[pl.ds(s * ZPT, ZPT)],
                        out_hbm.at[c, p, pl.ds(s * ZPT, ZPT)])
        if p + 1 < NP:
            for j in range(ZPT // CH):
                pltpu.sync_copy(zbuf, acc.at[pl.ds(s * ZPT + j * CH, CH)])
            plsc.subcore_barrier()


NH = N // 2
NP2 = N2 // 2


def _prep_body(x_ref, w1d_ref, deg2_ref, g_ref, gf_ref, dinv_ref):
    deg = deg2_ref[0] + deg2_ref[1] + 1.0
    dinv2 = lax.rsqrt(deg)
    d128 = jnp.concatenate(
        [jnp.broadcast_to(dinv2[:, 0:1], (NH, D_H)),
         jnp.broadcast_to(dinv2[:, 1:2], (NH, D_H))], axis=1)
    h = jnp.dot(x_ref[...], w1d_ref[...], preferred_element_type=jnp.float32)
    gf = h * d128
    g_ref[...] = gf.astype(jnp.bfloat16)
    gf_ref[...] = gf
    dinv_ref[...] = d128


def _psum(part_ref):
    acc = jnp.zeros((NH, 2 * D_H), jnp.float32)
    for c in range(NC):
        for p in range(NP):
            acc = acc + part_ref[c, p, :NH, :].astype(jnp.float32)
    return acc


def _bn(x, gm, bt):
    mu128 = jnp.mean(x, axis=0, keepdims=True)
    e2128 = jnp.mean(x * x, axis=0, keepdims=True)
    mu = 0.5 * (mu128[:, :D_H] + mu128[:, D_H:])
    e2 = 0.5 * (e2128[:, :D_H] + e2128[:, D_H:])
    var = e2 - mu * mu
    mu_b = jnp.concatenate([mu, mu], axis=1)
    var_b = jnp.concatenate([var, var], axis=1)
    return (x - mu_b) * lax.rsqrt(var_b + 1e-5) * gm + bt


def _mid_body(part_ref, g1_ref, dinv_ref, b1_ref, gm1_ref, bt1_ref, w2d_ref,
              g2_ref, g2f_ref):
    out1 = (_psum(part_ref) + g1_ref[...]) * dinv_ref[...] + b1_ref[...]
    hrelu = jnp.maximum(_bn(out1, gm1_ref[...], bt1_ref[...]), 0.0)
    h2 = jnp.dot(hrelu, w2d_ref[...], preferred_element_type=jnp.float32)
    g2f = h2 * dinv_ref[...]
    g2_ref[...] = g2f.astype(jnp.bfloat16)
    g2f_ref[...] = g2f


def _fin_body(part_ref, g2_ref, dinv_ref, b2_ref, gm2_ref, bt2_ref, out_ref):
    out2 = (_psum(part_ref) + g2_ref[...]) * dinv_ref[...] + b2_ref[...]
    out_ref[...] = _bn(out2, gm2_ref[...], bt2_ref[...])


_prep_call = pl.pallas_call(
    _prep_body,
    out_shape=[jax.ShapeDtypeStruct((NH, 2 * D_H), jnp.bfloat16),
               jax.ShapeDtypeStruct((NH, 2 * D_H), jnp.float32),
               jax.ShapeDtypeStruct((NH, 2 * D_H), jnp.float32)],
)

_mid_call = pl.pallas_call(
    _mid_body,
    out_shape=[jax.ShapeDtypeStruct((NH, 2 * D_H), jnp.bfloat16),
               jax.ShapeDtypeStruct((NH, 2 * D_H), jnp.float32)],
)

_fin_call = pl.pallas_call(
    _fin_body,
    out_shape=jax.ShapeDtypeStruct((NH, 2 * D_H), jnp.float32),
)


def _blockdiag2(w):
    a, b = w.shape
    z = jnp.zeros((a, b), w.dtype)
    return jnp.concatenate(
        [jnp.concatenate([w, z], axis=1), jnp.concatenate([z, w], axis=1)],
        axis=0)


def kernel(x, edge_index, W1, b1, gamma1, beta1, W2, b2, gamma2, beta2):
    pad = EPAD - E
    src = jnp.concatenate([edge_index[0], jnp.zeros((pad,), jnp.int32)])
    dst = jnp.concatenate([edge_index[1], jnp.full((pad,), N, jnp.int32)])
    srcp = src.reshape(NW, K, CH)
    dstp = dst.reshape(NW, K, CH)

    ones_c = jnp.ones((CH,), jnp.float32)
    zz1 = jnp.zeros((ZPT,), jnp.float32)
    zz2 = jnp.zeros((CH, D_H), jnp.bfloat16)

    x2 = x.reshape(NH, 2 * D_IN)
    w1d = _blockdiag2(W1)
    w2d = _blockdiag2(W2)
    b1t = jnp.tile(b1, 2)[None, :]
    gm1t = jnp.tile(gamma1, 2)[None, :]
    bt1t = jnp.tile(beta1, 2)[None, :]
    b2t = jnp.tile(b2, 2)[None, :]
    gm2t = jnp.tile(gamma2, 2)[None, :]
    bt2t = jnp.tile(beta2, 2)[None, :]

    degp = _deg_kernel(dstp, ones_c, zz1)
    deg2 = degp[:, :N].reshape(NC, NH, 2)

    g1, g1f, dinv = _prep_call(x2, w1d, deg2)

    part1 = _edge_kernel(g1.reshape(N, D_H), srcp, dstp, zz2)
    g2, g2f = _mid_call(part1.reshape(NC, NP, NP2, 2 * D_H), g1f, dinv,
                        b1t, gm1t, bt1t, w2d)

    part2 = _edge_kernel(g2.reshape(N, D_H), srcp, dstp, zz2)
    out = _fin_call(part2.reshape(NC, NP, NP2, 2 * D_H), g2f, dinv,
                    b2t, gm2t, bt2t)
    return out.reshape(N, D_H)
